# Initial kernel scaffold; baseline (speedup 1.0000x reference)
import jax
import jax.numpy as jnp
from jax.experimental import pallas as pl


def kernel(x, edge_index, W1, b1, W2, b2):
    raise NotImplementedError("write your pallas kernel here")


# speedup vs baseline: 32.6208x; 32.6208x over previous
import dataclasses
import functools

import jax
import jax.numpy as jnp
from jax import lax
from jax.experimental import pallas as pl
from jax.experimental.pallas import tpu as pltpu
from jax.experimental.pallas import tpu_sc as plsc

N = 10000
NPAD = 10112
E = 320000
EPAD = 327680
D1 = 16
NCLS = 40

NC, NS, L = 2, 16, 16
CPT = 2
NPART = 4
EQ = EPAD // NPART
CHK = 8192
NCHUNK = EQ // CHK
NZ = NPAD // L

_mesh = plsc.VectorSubcoreMesh(core_axis_name="c", subcore_axis_name="s")

_sc_params = pltpu.CompilerParams()
if "needs_layout_passes" in pltpu.CompilerParams.__dataclass_fields__:
    _sc_params = dataclasses.replace(_sc_params, needs_layout_passes=False)


def _deg_kernel_fn():
    ept = EPAD // (NC * NS)

    @functools.partial(
        pl.kernel, mesh=_mesh,
        out_type=jax.ShapeDtypeStruct((NC * NS * NPAD,), jnp.float32),
        compiler_params=_sc_params,
        scratch_types=[
            pltpu.VMEM((ept,), jnp.int32),
            pltpu.VMEM((NPAD,), jnp.float32),
        ])
    def k(dst_hbm, out_hbm, dst_v, acc_v):
        cid = lax.axis_index("c")
        sid = lax.axis_index("s")
        wid = cid * NS + sid
        pltpu.sync_copy(dst_hbm.at[pl.ds(wid * ept, ept)], dst_v)

        @pl.loop(0, NZ)
        def _(i):
            acc_v[pl.ds(i * L, L)] = jnp.zeros((L,), jnp.float32)

        ones = jnp.ones((L,), jnp.float32)

        @pl.loop(0, ept // L)
        def _(i):
            idx = dst_v[pl.ds(i * L, L)]
            plsc.addupdate_scatter(acc_v, [idx], ones)

        pltpu.sync_copy(acc_v, out_hbm.at[pl.ds(wid * NPAD, NPAD)])

    return k


def _msg_kernel_fn():
    @functools.partial(
        pl.kernel, mesh=_mesh,
        out_type=jax.ShapeDtypeStruct((NPART * D1 * NPAD,), jnp.float32),
        compiler_params=_sc_params,
        scratch_types=[
            pltpu.VMEM((CHK,), jnp.int32),
            pltpu.VMEM((CHK,), jnp.int32),
            pltpu.VMEM((NPAD,), jnp.float32),
            pltpu.VMEM((NPAD,), jnp.float32),
            pltpu.VMEM((NPAD,), jnp.float32),
            pltpu.VMEM((NPAD,), jnp.float32),
        ])
    def k(g_hbm, src_hbm, dst_hbm, out_hbm,
          src_v, dst_v, g0, g1, a0, a1):
        cid = lax.axis_index("c")
        sid = lax.axis_index("s")
        grp = NS // CPT
        col = CPT * (sid % grp)
        quarter = cid * CPT + sid // grp
        ebase = quarter * EQ
        pltpu.sync_copy(g_hbm.at[pl.ds(col * NPAD, NPAD)], g0)
        pltpu.sync_copy(g_hbm.at[pl.ds((col + 1) * NPAD, NPAD)], g1)

        @pl.loop(0, NZ)
        def _(i):
            a0[pl.ds(i * L, L)] = jnp.zeros((L,), jnp.float32)
            a1[pl.ds(i * L, L)] = jnp.zeros((L,), jnp.float32)

        @pl.loop(0, NCHUNK)
        def _(c):
            off = ebase + c * CHK
            pltpu.sync_copy(src_hbm.at[pl.ds(off, CHK)], src_v)
            pltpu.sync_copy(dst_hbm.at[pl.ds(off, CHK)], dst_v)

            @pl.loop(0, CHK // L)
            def _(i):
                s_vec = src_v[pl.ds(i * L, L)]
                d_vec = dst_v[pl.ds(i * L, L)]
                plsc.addupdate_scatter(a0, [d_vec],
                                       plsc.load_gather(g0, [s_vec]))
                plsc.addupdate_scatter(a1, [d_vec],
                                       plsc.load_gather(g1, [s_vec]))

        obase = (quarter * D1 + col) * NPAD
        pltpu.sync_copy(a0, out_hbm.at[pl.ds(obase, NPAD)])
        pltpu.sync_copy(a1, out_hbm.at[pl.ds(obase + NPAD, NPAD)])

    return k


_deg_kernel = _deg_kernel_fn()
_msg_kernel = _msg_kernel_fn()


def _tc_h1(w1t, xt):
    def body(w_ref, x_ref, o_ref):
        o_ref[...] = jnp.dot(w_ref[...], x_ref[...],
                             preferred_element_type=jnp.float32)

    return pl.pallas_call(
        body,
        out_shape=jax.ShapeDtypeStruct((D1, NPAD), jnp.float32),
    )(w1t, xt)


def _tc_norm_scale(degp, h1t):
    def body(degp_ref, h1_ref, g1_ref, dis_ref):
        deg = jnp.sum(degp_ref[...], axis=0, keepdims=True) + 1.0
        dis = lax.rsqrt(deg)
        dis_ref[...] = dis
        g1_ref[...] = dis * h1_ref[...]

    return pl.pallas_call(
        body,
        out_shape=(jax.ShapeDtypeStruct((D1, NPAD), jnp.float32),
                   jax.ShapeDtypeStruct((1, NPAD), jnp.float32)),
    )(degp, h1t)


def _tc_layer1_combine(acc1p, g1t, dis, b1):
    def body(acc_ref, g1_ref, dis_ref, b1_ref, q_ref):
        acc = (acc_ref[0] + acc_ref[1]) + (acc_ref[2] + acc_ref[3])
        s = dis_ref[...] * (acc + g1_ref[...]) + b1_ref[...][:, None]
        q_ref[...] = dis_ref[...] * jnp.maximum(s, 0.0)

    return pl.pallas_call(
        body,
        out_shape=jax.ShapeDtypeStruct((D1, NPAD), jnp.float32),
    )(acc1p, g1t, dis, b1)


def _tc_final(acc2p, qt, dis, W2, b2):
    def body(acc_ref, q_ref, dis_ref, w2_ref, b2_ref, o_ref):
        acc = (acc_ref[0] + acc_ref[1]) + (acc_ref[2] + acc_ref[3])
        t = dis_ref[...] * (acc + q_ref[...])
        res = lax.dot_general(t, w2_ref[...], (((0,), (0,)), ((), ())),
                              preferred_element_type=jnp.float32)
        o_ref[...] = res[0:N, :] + b2_ref[...]

    return pl.pallas_call(
        body,
        out_shape=jax.ShapeDtypeStruct((N, NCLS), jnp.float32),
    )(acc2p, qt, dis, W2, b2)


def kernel(x, edge_index, W1, b1, W2, b2):
    ei = edge_index.astype(jnp.int32)
    pad = jnp.full((2, EPAD - E), N, jnp.int32)
    ei = jnp.concatenate([ei, pad], axis=1)
    src, dst = ei[0], ei[1]
    xt = jnp.concatenate(
        [x.T, jnp.zeros((x.shape[1], NPAD - N), x.dtype)], axis=1)

    degp = _deg_kernel(dst).reshape(NC * NS, NPAD)
    h1t = _tc_h1(W1.T, xt)
    g1t, dis = _tc_norm_scale(degp, h1t)
    acc1p = _msg_kernel(g1t.reshape(-1), src, dst)
    qt = _tc_layer1_combine(
        acc1p.reshape(NPART, D1, NPAD), g1t, dis, b1)
    acc2p = _msg_kernel(qt.reshape(-1), src, dst)
    return _tc_final(
        acc2p.reshape(NPART, D1, NPAD), qt, dis, W2, b2)
